# baseline (speedup 1.0000x reference)
import functools

import jax
import jax.numpy as jnp
import numpy as np
from jax import lax
from jax.experimental import pallas as pl
from jax.experimental.pallas import tpu as pltpu
from jax.experimental.pallas import tpu_sc as plsc

MAX_HOP = 4
EMBED = 64
HIDDEN = 256
OUT = 128
N = 50000
E = 800000
B = 2

NC = 2
NS = 16
L = 16

HALF = N // 2
HWORDS = HALF * EMBED
CHUNK = 2000
EPT = E // NS
NCHUNK = EPT // CHUNK
ROWS16 = CHUNK // L

BSH = 13
BANDW = 1 << BSH
NB = (HWORDS + BANDW - 1) >> BSH
NKEY = NB + 1
NROUND = (NB + NS - 1) // NS
LASTBW = HWORDS - (NB - 1) * BANDW
SLICE = 50048
OFFW = 208
SENT = 0x40000000
WIN = 128

_SQRT_HALF = 0.7071067811865476


_AW = 512
_ANB = (N + _AW - 1) // _AW


def _argmax_body(p_ref, o_ref):
    o_ref[0, 0] = jnp.argmax(p_ref[...], axis=0).astype(jnp.int32)


_argmax_call = pl.pallas_call(
    _argmax_body,
    grid=(_ANB,),
    in_specs=[pl.BlockSpec((NCELLS := 64, _AW), lambda i: (0, i))],
    out_specs=pl.BlockSpec((1, 1, _AW), lambda i: (i, 0, 0)),
    out_shape=jax.ShapeDtypeStruct((_ANB, 1, _AW), jnp.int32),
)


def _sc_hist_body(edges_hbm, idx_hbm, c_hbm,
                  row_v, col_v, cell_v, cnt_v, starts_v, part_v, band_v,
                  offs_v, win_v, idxst_v, part_sh, offs_sh, idx_sh):
    cid = lax.axis_index("c")
    sid = lax.axis_index("s")
    iota = lax.iota(jnp.int32, L)
    ones_f = jnp.full((L,), 1.0, jnp.float32)
    ones_i = jnp.full((L,), 1, jnp.int32)
    zero_f = jnp.zeros((L,), jnp.float32)

    @pl.when(sid < 10)
    def _():
        pltpu.sync_copy(idx_hbm.at[pl.ds(sid * 5000, 5000)], idxst_v)
        pltpu.sync_copy(idxst_v, idx_sh.at[pl.ds(sid * 5000, 5000)])
    plsc.subcore_barrier()

    cbase = cid * HWORDS

    def _key_of(j):
        sl = pl.ds(j * L, L)
        rel = row_v[sl] * EMBED + cell_v[sl] - cbase
        ok = (rel >= 0) & (rel < HWORDS)
        rel = jnp.where(ok, rel, jnp.int32(NB << BSH))
        return rel, (rel >> BSH) * L + iota

    for g in range(B):
        def _zcnt(j, _):
            cnt_v[pl.ds(j * L, L)] = jnp.zeros((L,), jnp.int32)
            return 0
        lax.fori_loop(0, OFFW, _zcnt, 0)

        def _sentinel(j, _):
            part_v[pl.ds(j * L, L)] = jnp.full((L,), SENT, jnp.int32)
            return 0
        lax.fori_loop(0, SLICE // L, _sentinel, 0)

        def _chunk1(ch, _):
            pltpu.sync_copy(edges_hbm.at[g, 0, sid, ch], row_v)
            pltpu.sync_copy(edges_hbm.at[g, 1, sid, ch], col_v)
            pltpu.sync_copy(idx_sh.at[col_v], cell_v)

            def _lanes(j, _):
                _, ci = _key_of(j)
                plsc.addupdate_scatter(cnt_v, [ci], ones_i)
                return 0
            lax.fori_loop(0, ROWS16, _lanes, 0)
            return 0
        lax.fori_loop(0, NCHUNK, _chunk1, 0)

        def _prefix(k, carry):
            v = cnt_v[pl.ds(k * L, L)]
            ex = plsc.cumsum(v) - v + carry
            cnt_v[pl.ds(k * L, L)] = ex
            return carry + jnp.sum(v)
        lax.fori_loop(0, NKEY, _prefix, jnp.int32(0))

        def _starts(kg, _):
            starts_v[pl.ds(kg * L, L)] = plsc.load_gather(
                cnt_v, [(kg * L + iota) * L])
            return 0
        lax.fori_loop(0, OFFW // L, _starts, 0)

        def _chunk2(ch, _):
            pltpu.sync_copy(edges_hbm.at[g, 0, sid, ch], row_v)
            pltpu.sync_copy(edges_hbm.at[g, 1, sid, ch], col_v)
            pltpu.sync_copy(idx_sh.at[col_v], cell_v)

            def _lanes(j, _):
                rel, ci = _key_of(j)
                pos = plsc.load_gather(cnt_v, [ci])
                plsc.store_scatter(part_v, [pos], rel)
                plsc.addupdate_scatter(cnt_v, [ci], ones_i)
                return 0
            lax.fori_loop(0, ROWS16, _lanes, 0)
            return 0
        lax.fori_loop(0, NCHUNK, _chunk2, 0)

        pltpu.sync_copy(part_v, part_sh.at[pl.ds(sid * SLICE, SLICE)])
        pltpu.sync_copy(starts_v, offs_sh.at[pl.ds(sid * OFFW, OFFW)])
        plsc.subcore_barrier()

        pltpu.sync_copy(offs_sh, offs_v)
        gbase = g * N * EMBED + cbase

        def _round(r, _):
            m = r * NS + sid

            @pl.when(m < NB)
            def _apply():
                def _zb(j, _):
                    band_v[pl.ds(j * L, L)] = zero_f
                    return 0
                lax.fori_loop(0, (BANDW + L) // L, _zb, 0)
                mbase = m << BSH

                def _sender(u, _):
                    oidx = u * OFFW + m + jnp.minimum(iota, 1)
                    se = plsc.load_gather(offs_v, [oidx])
                    s = se[0]
                    e = se[1]
                    s8 = pl.multiple_of(s & ~7, 8)
                    nw = (e - s8 + WIN - 1) // WIN

                    def _win(w, _):
                        off = pl.multiple_of(u * SLICE + s8 + w * WIN, 8)
                        pltpu.sync_copy(part_sh.at[pl.ds(off, WIN)], win_v)

                        def _wl(j, _):
                            x = win_v[pl.ds(j * L, L)] - mbase
                            ok = (x >= 0) & (x < BANDW)
                            plsc.addupdate_scatter(
                                band_v, [jnp.where(ok, x, BANDW)], ones_f)
                            return 0
                        lax.fori_loop(0, WIN // L, _wl, 0)
                        return 0
                    lax.fori_loop(0, nw, _win, 0)
                    return 0
                lax.fori_loop(0, NS, _sender, 0)

                hoff = pl.multiple_of(gbase + mbase, 8)

                @pl.when(m < NB - 1)
                def _():
                    pltpu.sync_copy(band_v.at[pl.ds(0, BANDW)],
                                    c_hbm.at[pl.ds(hoff, BANDW)])

                @pl.when(m == NB - 1)
                def _():
                    pltpu.sync_copy(band_v.at[pl.ds(0, LASTBW)],
                                    c_hbm.at[pl.ds(hoff, LASTBW)])
            return 0
        lax.fori_loop(0, NROUND, _round, 0)
        plsc.subcore_barrier()


@functools.cache
def _sc_hist_call():
    mesh = plsc.VectorSubcoreMesh(core_axis_name="c", subcore_axis_name="s")
    return pl.kernel(
        _sc_hist_body,
        mesh=mesh,
        out_type=jax.ShapeDtypeStruct((B * N * EMBED,), jnp.float32),
        compiler_params=pltpu.CompilerParams(needs_layout_passes=False),
        scratch_types=[
            pltpu.VMEM((CHUNK,), jnp.int32),
            pltpu.VMEM((CHUNK,), jnp.int32),
            pltpu.VMEM((CHUNK,), jnp.int32),
            pltpu.VMEM((OFFW * L,), jnp.int32),
            pltpu.VMEM((OFFW,), jnp.int32),
            pltpu.VMEM((SLICE,), jnp.int32),
            pltpu.VMEM((BANDW + L,), jnp.float32),
            pltpu.VMEM((NS * OFFW,), jnp.int32),
            pltpu.VMEM((WIN,), jnp.int32),
            pltpu.VMEM((5000,), jnp.int32),
            pltpu.VMEM_SHARED((NS * SLICE,), jnp.int32),
            pltpu.VMEM_SHARED((NS * OFFW,), jnp.int32),
            pltpu.VMEM_SHARED((N,), jnp.int32),
        ],
    )


def _prep_body(e_ref, s_ref, w1_ref, m_ref):
    rep = jnp.dot(e_ref[...], s_ref[...],
                  preferred_element_type=jnp.float32,
                  precision=lax.Precision.HIGHEST)
    m_ref[...] = jnp.dot(rep, w1_ref[...],
                         preferred_element_type=jnp.float32,
                         precision=lax.Precision.HIGHEST)


_prep_call = pl.pallas_call(
    _prep_body,
    out_shape=jax.ShapeDtypeStruct((EMBED, HIDDEN), jnp.float32),
)

_S_REP = np.kron(np.eye(EMBED, dtype=np.float32),
                 np.ones((1, MAX_HOP), dtype=np.float32))


_R = 1000
_RNB = N // _R


def _mlp_body(c_ref, m_ref, p_ref, w2_ref, o_ref):
    h = jnp.dot(c_ref[0], m_ref[...],
                preferred_element_type=jnp.float32,
                precision=lax.Precision.HIGHEST) + p_ref[0]
    mu = jnp.mean(h, axis=-1, keepdims=True)
    var = jnp.mean((h - mu) ** 2, axis=-1, keepdims=True)
    h = (h - mu) * lax.rsqrt(var + 1e-5) * p_ref[1] + p_ref[2]
    h = h * 0.5 * (1.0 + lax.erf(h * _SQRT_HALF))
    o_ref[0] = jnp.dot(h, w2_ref[...],
                       preferred_element_type=jnp.float32,
                       precision=lax.Precision.HIGHEST) + p_ref[3, :OUT]


_mlp_call = pl.pallas_call(
    _mlp_body,
    grid=(B, _RNB),
    in_specs=[
        pl.BlockSpec((1, _R, EMBED), lambda g, i: (g, i, 0)),
        pl.BlockSpec((EMBED, HIDDEN), lambda g, i: (0, 0)),
        pl.BlockSpec((8, HIDDEN), lambda g, i: (0, 0)),
        pl.BlockSpec((HIDDEN, OUT), lambda g, i: (0, 0)),
    ],
    out_specs=pl.BlockSpec((1, _R, OUT), lambda g, i: (g, i, 0)),
    out_shape=jax.ShapeDtypeStruct((B, N, OUT), jnp.float32),
)


def kernel(edge_index_list, num_nodes_list, perturb_one_hot, emb_table,
           W1, b1, gamma, beta, W2, b2):
    del num_nodes_list

    idx = _argmax_call(perturb_one_hot).reshape(-1)[:N]

    edges5 = edge_index_list.reshape(B, 2, NS, NCHUNK, CHUNK)
    c = _sc_hist_call()(edges5, idx).reshape(B, N, EMBED)

    m = _prep_call(emb_table, _S_REP, W1)
    params = jnp.zeros((8, HIDDEN), jnp.float32)
    params = params.at[0].set(b1).at[1].set(gamma).at[2].set(beta)
    params = params.at[3, :OUT].set(b2)

    return _mlp_call(c, m, params, W2)


# speedup vs baseline: 1.0412x; 1.0412x over previous
import functools

import jax
import jax.numpy as jnp
import numpy as np
from jax import lax
from jax.experimental import pallas as pl
from jax.experimental.pallas import tpu as pltpu
from jax.experimental.pallas import tpu_sc as plsc

MAX_HOP = 4
EMBED = 64
HIDDEN = 256
OUT = 128
N = 50000
E = 800000
B = 2

NC = 2
NS = 16
L = 16

HALF = N // 2
HWORDS = HALF * EMBED
CHUNK = 2000
EPT = E // NS
NCHUNK = EPT // CHUNK
ROWS16 = CHUNK // L

BSH = 13
BANDW = 1 << BSH
NB = (HWORDS + BANDW - 1) >> BSH
NKEY = NB + 1
NROUND = (NB + NS - 1) // NS
LASTBW = HWORDS - (NB - 1) * BANDW
SLICE = 50048
OFFW = 208
SENT = 0x40000000
WIN = 128

_SQRT_HALF = 0.7071067811865476


_AW = 512
_ANB = (N + _AW - 1) // _AW


def _argmax_body(p_ref, o_ref):
    o_ref[0, 0] = jnp.argmax(p_ref[...], axis=0).astype(jnp.int32)


_argmax_call = pl.pallas_call(
    _argmax_body,
    grid=(_ANB,),
    in_specs=[pl.BlockSpec((NCELLS := 64, _AW), lambda i: (0, i))],
    out_specs=pl.BlockSpec((1, 1, _AW), lambda i: (i, 0, 0)),
    out_shape=jax.ShapeDtypeStruct((_ANB, 1, _AW), jnp.int32),
)


def _sc_hist_body(edges_hbm, idx_hbm, c_hbm,
                  row_v, col_v, cell_v, cnt_v, starts_v, part_v, band_v,
                  offs_v, win16_v, idxst_v, part_sh, offs_sh, idx_sh, sem):
    cid = lax.axis_index("c")
    sid = lax.axis_index("s")
    iota = lax.iota(jnp.int32, L)
    ones_f = jnp.full((L,), 1.0, jnp.float32)
    ones_i = jnp.full((L,), 1, jnp.int32)
    zero_f = jnp.zeros((L,), jnp.float32)

    @pl.when(sid < 12)
    def _():
        pltpu.sync_copy(idx_hbm.at[pl.ds(sid * 4000, 4000)], idxst_v)
        pltpu.sync_copy(idxst_v, idx_sh.at[pl.ds(sid * 4000, 4000)])

    @pl.when(sid == 12)
    def _():
        pltpu.sync_copy(idx_hbm.at[pl.ds(48000, 2000)],
                        idxst_v.at[pl.ds(0, 2000)])
        pltpu.sync_copy(idxst_v.at[pl.ds(0, 2000)],
                        idx_sh.at[pl.ds(48000, 2000)])
    plsc.subcore_barrier()

    cbase = cid * HWORDS

    def _key_of(j):
        sl = pl.ds(j * L, L)
        rel = row_v[sl] * EMBED + cell_v[sl] - cbase
        ok = (rel >= 0) & (rel < HWORDS)
        rel = jnp.where(ok, rel, jnp.int32(NB << BSH))
        return rel, (rel >> BSH) * L + iota

    for g in range(B):
        def _zcnt(j, _):
            cnt_v[pl.ds(j * L, L)] = jnp.zeros((L,), jnp.int32)
            return 0
        lax.fori_loop(0, OFFW, _zcnt, 0)

        def _sentinel(j, _):
            part_v[pl.ds(j * L, L)] = jnp.full((L,), SENT, jnp.int32)
            return 0
        lax.fori_loop(0, SLICE // L, _sentinel, 0)

        def _chunk1(ch, _):
            pltpu.sync_copy(edges_hbm.at[g, 0, sid, ch], row_v)
            pltpu.sync_copy(edges_hbm.at[g, 1, sid, ch], col_v)
            pltpu.sync_copy(idx_sh.at[col_v], cell_v)

            def _lanes(j, _):
                _, ci = _key_of(j)
                plsc.addupdate_scatter(cnt_v, [ci], ones_i)
                return 0
            lax.fori_loop(0, ROWS16, _lanes, 0)
            return 0
        lax.fori_loop(0, NCHUNK, _chunk1, 0)

        def _prefix(k, carry):
            v = cnt_v[pl.ds(k * L, L)]
            ex = plsc.cumsum(v) - v + carry
            cnt_v[pl.ds(k * L, L)] = ex
            return carry + jnp.sum(v)
        lax.fori_loop(0, NKEY, _prefix, jnp.int32(0))

        def _starts(kg, _):
            starts_v[pl.ds(kg * L, L)] = plsc.load_gather(
                cnt_v, [(kg * L + iota) * L])
            return 0
        lax.fori_loop(0, OFFW // L, _starts, 0)

        def _chunk2(ch, _):
            pltpu.sync_copy(edges_hbm.at[g, 0, sid, ch], row_v)
            pltpu.sync_copy(edges_hbm.at[g, 1, sid, ch], col_v)
            pltpu.sync_copy(idx_sh.at[col_v], cell_v)

            def _lanes(j, _):
                rel, ci = _key_of(j)
                pos = plsc.load_gather(cnt_v, [ci])
                plsc.store_scatter(part_v, [pos], rel)
                plsc.addupdate_scatter(cnt_v, [ci], ones_i)
                return 0
            lax.fori_loop(0, ROWS16, _lanes, 0)
            return 0
        lax.fori_loop(0, NCHUNK, _chunk2, 0)

        pltpu.sync_copy(part_v, part_sh.at[pl.ds(sid * SLICE, SLICE)])
        pltpu.sync_copy(starts_v, offs_sh.at[pl.ds(sid * OFFW, OFFW)])
        plsc.subcore_barrier()

        pltpu.sync_copy(offs_sh, offs_v)
        gbase = g * N * EMBED + cbase

        def _round(r, _):
            m = r * NS + sid

            @pl.when(m < NB)
            def _apply():
                def _zb(j, _):
                    band_v[pl.ds(j * L, L)] = zero_f
                    return 0
                lax.fori_loop(0, (BANDW + L) // L, _zb, 0)
                mbase = m << BSH

                s_vec = plsc.load_gather(offs_v, [iota * OFFW + m])
                e_vec = plsc.load_gather(offs_v, [iota * OFFW + m + 1])
                s8_vec = s_vec & ~7

                copies = []
                for u in range(NS):
                    off = pl.multiple_of(u * SLICE + s8_vec[u], 8)
                    copies.append(pltpu.async_copy(
                        part_sh.at[pl.ds(off, WIN)],
                        win16_v.at[pl.ds(u * WIN, WIN)], sem))

                def _consume(buf_off, mb):
                    def _wl(j, _):
                        x = win16_v[pl.ds(buf_off + j * L, L)] - mb
                        ok = (x >= 0) & (x < BANDW)
                        plsc.addupdate_scatter(
                            band_v, [jnp.where(ok, x, BANDW)], ones_f)
                        return 0
                    lax.fori_loop(0, WIN // L, _wl, 0)

                for u in range(NS):
                    copies[u].wait()
                    _consume(u * WIN, mbase)
                    s8 = pl.multiple_of(s8_vec[u], 8)
                    nw = (e_vec[u] - s8 + WIN - 1) // WIN

                    def _win(w, _):
                        off = pl.multiple_of(u * SLICE + s8 + w * WIN, 8)
                        pltpu.sync_copy(
                            part_sh.at[pl.ds(off, WIN)],
                            win16_v.at[pl.ds(u * WIN, WIN)])
                        _consume(u * WIN, mbase)
                        return 0
                    lax.fori_loop(1, nw, _win, 0)

                hoff = pl.multiple_of(gbase + mbase, 8)

                @pl.when(m < NB - 1)
                def _():
                    pltpu.sync_copy(band_v.at[pl.ds(0, BANDW)],
                                    c_hbm.at[pl.ds(hoff, BANDW)])

                @pl.when(m == NB - 1)
                def _():
                    pltpu.sync_copy(band_v.at[pl.ds(0, LASTBW)],
                                    c_hbm.at[pl.ds(hoff, LASTBW)])
            return 0
        lax.fori_loop(0, NROUND, _round, 0)
        plsc.subcore_barrier()


@functools.cache
def _sc_hist_call():
    mesh = plsc.VectorSubcoreMesh(core_axis_name="c", subcore_axis_name="s")
    return pl.kernel(
        _sc_hist_body,
        mesh=mesh,
        out_type=jax.ShapeDtypeStruct((B * N * EMBED,), jnp.float32),
        compiler_params=pltpu.CompilerParams(needs_layout_passes=False),
        scratch_types=[
            pltpu.VMEM((CHUNK,), jnp.int32),
            pltpu.VMEM((CHUNK,), jnp.int32),
            pltpu.VMEM((CHUNK,), jnp.int32),
            pltpu.VMEM((OFFW * L,), jnp.int32),
            pltpu.VMEM((OFFW,), jnp.int32),
            pltpu.VMEM((SLICE,), jnp.int32),
            pltpu.VMEM((BANDW + L,), jnp.float32),
            pltpu.VMEM((NS * OFFW,), jnp.int32),
            pltpu.VMEM((NS * WIN,), jnp.int32),
            pltpu.VMEM((4000,), jnp.int32),
            pltpu.VMEM_SHARED((NS * SLICE,), jnp.int32),
            pltpu.VMEM_SHARED((NS * OFFW,), jnp.int32),
            pltpu.VMEM_SHARED((N,), jnp.int32),
            pltpu.SemaphoreType.DMA,
        ],
    )


def _prep_body(e_ref, s_ref, w1_ref, m_ref):
    rep = jnp.dot(e_ref[...], s_ref[...],
                  preferred_element_type=jnp.float32,
                  precision=lax.Precision.HIGHEST)
    m_ref[...] = jnp.dot(rep, w1_ref[...],
                         preferred_element_type=jnp.float32,
                         precision=lax.Precision.HIGHEST)


_prep_call = pl.pallas_call(
    _prep_body,
    out_shape=jax.ShapeDtypeStruct((EMBED, HIDDEN), jnp.float32),
)

_S_REP = np.kron(np.eye(EMBED, dtype=np.float32),
                 np.ones((1, MAX_HOP), dtype=np.float32))


_R = 1000
_RNB = N // _R


def _mlp_body(c_ref, m_ref, p_ref, w2_ref, o_ref):
    h = jnp.dot(c_ref[0], m_ref[...],
                preferred_element_type=jnp.float32,
                precision=lax.Precision.HIGHEST) + p_ref[0]
    mu = jnp.mean(h, axis=-1, keepdims=True)
    var = jnp.mean((h - mu) ** 2, axis=-1, keepdims=True)
    h = (h - mu) * lax.rsqrt(var + 1e-5) * p_ref[1] + p_ref[2]
    h = h * 0.5 * (1.0 + lax.erf(h * _SQRT_HALF))
    o_ref[0] = jnp.dot(h, w2_ref[...],
                       preferred_element_type=jnp.float32,
                       precision=lax.Precision.HIGHEST) + p_ref[3, :OUT]


_mlp_call = pl.pallas_call(
    _mlp_body,
    grid=(B, _RNB),
    in_specs=[
        pl.BlockSpec((1, _R, EMBED), lambda g, i: (g, i, 0)),
        pl.BlockSpec((EMBED, HIDDEN), lambda g, i: (0, 0)),
        pl.BlockSpec((8, HIDDEN), lambda g, i: (0, 0)),
        pl.BlockSpec((HIDDEN, OUT), lambda g, i: (0, 0)),
    ],
    out_specs=pl.BlockSpec((1, _R, OUT), lambda g, i: (g, i, 0)),
    out_shape=jax.ShapeDtypeStruct((B, N, OUT), jnp.float32),
)


def kernel(edge_index_list, num_nodes_list, perturb_one_hot, emb_table,
           W1, b1, gamma, beta, W2, b2):
    del num_nodes_list

    idx = _argmax_call(perturb_one_hot).reshape(-1)[:N]

    edges5 = edge_index_list.reshape(B, 2, NS, NCHUNK, CHUNK)
    c = _sc_hist_call()(edges5, idx).reshape(B, N, EMBED)

    m = _prep_call(emb_table, _S_REP, W1)
    params = jnp.zeros((8, HIDDEN), jnp.float32)
    params = params.at[0].set(b1).at[1].set(gamma).at[2].set(beta)
    params = params.at[3, :OUT].set(b2)

    return _mlp_call(c, m, params, W2)


# speedup vs baseline: 1.2822x; 1.2314x over previous
import functools

import jax
import jax.numpy as jnp
import numpy as np
from jax import lax
from jax.experimental import pallas as pl
from jax.experimental.pallas import tpu as pltpu
from jax.experimental.pallas import tpu_sc as plsc

MAX_HOP = 4
EMBED = 64
HIDDEN = 256
OUT = 128
N = 50000
E = 800000
B = 2

NC = 2
NS = 16
L = 16

HALF = N // 2
HWORDS = HALF * EMBED
CHUNK = 2000
EPT = E // NS
NCHUNK = EPT // CHUNK
ROWS16 = CHUNK // L

GWORDS = N * EMBED
BSH = 13
BANDW = 1 << BSH
NB = (GWORDS + BANDW - 1) >> BSH
NKEY = NB + 1
NROUND = (NB + NS - 1) // NS
LASTBW = GWORDS - (NB - 1) * BANDW
SLICE = 50048
OFFW = 400
SENT = 0x40000000
WIN = 128

_SQRT_HALF = 0.7071067811865476


_AW = 512
_ANB = (N + _AW - 1) // _AW


def _argmax_body(p_ref, o_ref):
    o_ref[0, 0] = jnp.argmax(p_ref[...], axis=0).astype(jnp.int32)


_argmax_call = pl.pallas_call(
    _argmax_body,
    grid=(_ANB,),
    in_specs=[pl.BlockSpec((NCELLS := 64, _AW), lambda i: (0, i))],
    out_specs=pl.BlockSpec((1, 1, _AW), lambda i: (i, 0, 0)),
    out_shape=jax.ShapeDtypeStruct((_ANB, 1, _AW), jnp.int32),
)


def _sc_hist_body(edges_hbm, idx_hbm, c_hbm,
                  row_v, col_v, cell_v, cnt_v, starts_v, part_v, band_v,
                  offs16_v, win16_v, idxst_v, part_sh, offs_sh, idx_sh, sem):
    cid = lax.axis_index("c")
    sid = lax.axis_index("s")
    iota = lax.iota(jnp.int32, L)
    ones_f = jnp.full((L,), 1.0, jnp.float32)
    ones_i = jnp.full((L,), 1, jnp.int32)
    zero_f = jnp.zeros((L,), jnp.float32)

    @pl.when(sid < 15)
    def _():
        pltpu.sync_copy(idx_hbm.at[pl.ds(sid * 3200, 3200)], idxst_v)
        pltpu.sync_copy(idxst_v, idx_sh.at[pl.ds(sid * 3200, 3200)])

    @pl.when(sid == 15)
    def _():
        pltpu.sync_copy(idx_hbm.at[pl.ds(48000, 2000)],
                        idxst_v.at[pl.ds(0, 2000)])
        pltpu.sync_copy(idxst_v.at[pl.ds(0, 2000)],
                        idx_sh.at[pl.ds(48000, 2000)])
    plsc.subcore_barrier()

    if True:
        g = cid

        def _key_of(j):
            sl = pl.ds(j * L, L)
            rel = row_v[sl] * EMBED + cell_v[sl]
            return rel, (rel >> BSH) * L + iota

        def _zcnt(j, _):
            cnt_v[pl.ds(j * L, L)] = jnp.zeros((L,), jnp.int32)
            return 0
        lax.fori_loop(0, OFFW, _zcnt, 0)

        def _sentinel(j, _):
            part_v[pl.ds(j * L, L)] = jnp.full((L,), SENT, jnp.int32)
            return 0
        lax.fori_loop(0, SLICE // L, _sentinel, 0)

        def _chunk1(ch, _):
            pltpu.sync_copy(edges_hbm.at[g, 0, sid, ch], row_v)
            pltpu.sync_copy(edges_hbm.at[g, 1, sid, ch], col_v)
            pltpu.sync_copy(idx_sh.at[col_v], cell_v)

            def _lanes(j, _):
                _, ci = _key_of(j)
                plsc.addupdate_scatter(cnt_v, [ci], ones_i)
                return 0
            lax.fori_loop(0, ROWS16, _lanes, 0)
            return 0
        lax.fori_loop(0, NCHUNK, _chunk1, 0)

        def _prefix(k, carry):
            v = cnt_v[pl.ds(k * L, L)]
            ex = plsc.cumsum(v) - v + carry
            cnt_v[pl.ds(k * L, L)] = ex
            return carry + jnp.sum(v)
        lax.fori_loop(0, NKEY, _prefix, jnp.int32(0))

        def _starts(kg, _):
            starts_v[pl.ds(kg * L, L)] = plsc.load_gather(
                cnt_v, [(kg * L + iota) * L])
            return 0
        lax.fori_loop(0, OFFW // L, _starts, 0)

        def _chunk2(ch, _):
            pltpu.sync_copy(edges_hbm.at[g, 0, sid, ch], row_v)
            pltpu.sync_copy(edges_hbm.at[g, 1, sid, ch], col_v)
            pltpu.sync_copy(idx_sh.at[col_v], cell_v)

            def _lanes(j, _):
                rel, ci = _key_of(j)
                pos = plsc.load_gather(cnt_v, [ci])
                plsc.store_scatter(part_v, [pos], rel)
                plsc.addupdate_scatter(cnt_v, [ci], ones_i)
                return 0
            lax.fori_loop(0, ROWS16, _lanes, 0)
            return 0
        lax.fori_loop(0, NCHUNK, _chunk2, 0)

        pltpu.sync_copy(part_v, part_sh.at[pl.ds(sid * SLICE, SLICE)])
        pltpu.sync_copy(starts_v, offs_sh.at[pl.ds(sid * OFFW, OFFW)])
        plsc.subcore_barrier()

        gbase = g * GWORDS

        def _round(r, _):
            m = r * NS + sid

            @pl.when(m < NB)
            def _apply():
                def _zb(j, _):
                    band_v[pl.ds(j * L, L)] = zero_f
                    return 0
                lax.fori_loop(0, (BANDW + L) // L, _zb, 0)
                mbase = m << BSH

                ocopies = []
                for u in range(NS):
                    fo = pl.multiple_of((u * OFFW + m) & ~7, 8)
                    ocopies.append(pltpu.async_copy(
                        offs_sh.at[pl.ds(fo, L)],
                        offs16_v.at[pl.ds(u * L, L)], sem))
                for u in range(NS):
                    ocopies[u].wait()
                a = m & 7
                s_vec = plsc.load_gather(offs16_v, [iota * L + a])
                e_vec = plsc.load_gather(offs16_v, [iota * L + a + 1])
                s8_vec = s_vec & ~7

                copies = []
                for u in range(NS):
                    off = pl.multiple_of(u * SLICE + s8_vec[u], 8)
                    copies.append(pltpu.async_copy(
                        part_sh.at[pl.ds(off, WIN)],
                        win16_v.at[pl.ds(u * WIN, WIN)], sem))

                def _consume(buf_off, mb):
                    def _wl(j, _):
                        x = win16_v[pl.ds(buf_off + j * L, L)] - mb
                        ok = (x >= 0) & (x < BANDW)
                        plsc.addupdate_scatter(
                            band_v, [jnp.where(ok, x, BANDW)], ones_f)
                        return 0
                    lax.fori_loop(0, WIN // L, _wl, 0)

                for u in range(NS):
                    copies[u].wait()
                    _consume(u * WIN, mbase)
                    s8 = pl.multiple_of(s8_vec[u], 8)
                    nw = (e_vec[u] - s8 + WIN - 1) // WIN

                    def _win(w, _):
                        off = pl.multiple_of(u * SLICE + s8 + w * WIN, 8)
                        pltpu.sync_copy(
                            part_sh.at[pl.ds(off, WIN)],
                            win16_v.at[pl.ds(u * WIN, WIN)])
                        _consume(u * WIN, mbase)
                        return 0
                    lax.fori_loop(1, nw, _win, 0)

                hoff = pl.multiple_of(gbase + mbase, 8)

                @pl.when(m < NB - 1)
                def _():
                    pltpu.sync_copy(band_v.at[pl.ds(0, BANDW)],
                                    c_hbm.at[pl.ds(hoff, BANDW)])

                @pl.when(m == NB - 1)
                def _():
                    pltpu.sync_copy(band_v.at[pl.ds(0, LASTBW)],
                                    c_hbm.at[pl.ds(hoff, LASTBW)])
            return 0
        lax.fori_loop(0, NROUND, _round, 0)
        plsc.subcore_barrier()


@functools.cache
def _sc_hist_call():
    mesh = plsc.VectorSubcoreMesh(core_axis_name="c", subcore_axis_name="s")
    return pl.kernel(
        _sc_hist_body,
        mesh=mesh,
        out_type=jax.ShapeDtypeStruct((B * N * EMBED,), jnp.float32),
        compiler_params=pltpu.CompilerParams(needs_layout_passes=False),
        scratch_types=[
            pltpu.VMEM((CHUNK,), jnp.int32),
            pltpu.VMEM((CHUNK,), jnp.int32),
            pltpu.VMEM((CHUNK,), jnp.int32),
            pltpu.VMEM((OFFW * L,), jnp.int32),
            pltpu.VMEM((OFFW,), jnp.int32),
            pltpu.VMEM((SLICE,), jnp.int32),
            pltpu.VMEM((BANDW + L,), jnp.float32),
            pltpu.VMEM((NS * L,), jnp.int32),
            pltpu.VMEM((NS * WIN,), jnp.int32),
            pltpu.VMEM((3200,), jnp.int32),
            pltpu.VMEM_SHARED((NS * SLICE,), jnp.int32),
            pltpu.VMEM_SHARED((NS * OFFW,), jnp.int32),
            pltpu.VMEM_SHARED((N,), jnp.int32),
            pltpu.SemaphoreType.DMA,
        ],
    )


def _prep_body(e_ref, s_ref, w1_ref, m_ref):
    rep = jnp.dot(e_ref[...], s_ref[...],
                  preferred_element_type=jnp.float32,
                  precision=lax.Precision.HIGHEST)
    m_ref[...] = jnp.dot(rep, w1_ref[...],
                         preferred_element_type=jnp.float32,
                         precision=lax.Precision.HIGHEST)


_prep_call = pl.pallas_call(
    _prep_body,
    out_shape=jax.ShapeDtypeStruct((EMBED, HIDDEN), jnp.float32),
)

_S_REP = np.kron(np.eye(EMBED, dtype=np.float32),
                 np.ones((1, MAX_HOP), dtype=np.float32))


_R = 1000
_RNB = N // _R


def _mlp_body(c_ref, m_ref, p_ref, w2_ref, o_ref):
    h = jnp.dot(c_ref[0], m_ref[...],
                preferred_element_type=jnp.float32,
                precision=lax.Precision.HIGHEST) + p_ref[0]
    mu = jnp.mean(h, axis=-1, keepdims=True)
    var = jnp.mean((h - mu) ** 2, axis=-1, keepdims=True)
    h = (h - mu) * lax.rsqrt(var + 1e-5) * p_ref[1] + p_ref[2]
    h = h * 0.5 * (1.0 + lax.erf(h * _SQRT_HALF))
    o_ref[0] = jnp.dot(h, w2_ref[...],
                       preferred_element_type=jnp.float32,
                       precision=lax.Precision.HIGHEST) + p_ref[3, :OUT]


_mlp_call = pl.pallas_call(
    _mlp_body,
    grid=(B, _RNB),
    in_specs=[
        pl.BlockSpec((1, _R, EMBED), lambda g, i: (g, i, 0)),
        pl.BlockSpec((EMBED, HIDDEN), lambda g, i: (0, 0)),
        pl.BlockSpec((8, HIDDEN), lambda g, i: (0, 0)),
        pl.BlockSpec((HIDDEN, OUT), lambda g, i: (0, 0)),
    ],
    out_specs=pl.BlockSpec((1, _R, OUT), lambda g, i: (g, i, 0)),
    out_shape=jax.ShapeDtypeStruct((B, N, OUT), jnp.float32),
)


def kernel(edge_index_list, num_nodes_list, perturb_one_hot, emb_table,
           W1, b1, gamma, beta, W2, b2):
    del num_nodes_list

    idx = _argmax_call(perturb_one_hot).reshape(-1)[:N]

    edges5 = edge_index_list.reshape(B, 2, NS, NCHUNK, CHUNK)
    c = _sc_hist_call()(edges5, idx).reshape(B, N, EMBED)

    m = _prep_call(emb_table, _S_REP, W1)
    params = jnp.zeros((8, HIDDEN), jnp.float32)
    params = params.at[0].set(b1).at[1].set(gamma).at[2].set(beta)
    params = params.at[3, :OUT].set(b2)

    return _mlp_call(c, m, params, W2)
